# baseline (speedup 1.0000x reference)
import functools

import numpy as np
import jax
import jax.numpy as jnp
from jax import lax
from jax.experimental import pallas as pl
from jax.experimental.pallas import tpu as pltpu
from jax.experimental.pallas import tpu_sc as plsc

N = 10000
E = 160000
IN = 8
SH = 4
OUT = 16
F = 64
H = 64
K = IN * SH
WID = 2 * OUT
SCALE = 1.0 / float(np.sqrt(IN * SH))

NC, NS = 2, 16
NW = NC * NS
EPW = E // NW
NPT = N // NS
CH = 500
NCH = EPW // CH
NGR = (CH + 15) // 16

BE = 640

_c = np.arange(K * OUT)
_RS2 = ((_c // 128)[None, :] == np.arange(SH)[:, None]).astype(np.float32)


def _sc_tp_scatter(g, dst, src2, table, zeros):
    mesh = plsc.VectorSubcoreMesh(core_axis_name="c", subcore_axis_name="s")

    @functools.partial(
        pl.kernel, mesh=mesh,
        out_type=jax.ShapeDtypeStruct((NC, N, WID), jnp.float32),
        compiler_params=pltpu.CompilerParams(use_tc_tiling_on_sc=False,
                                             needs_layout_passes=False),
        scratch_types=[
            pltpu.VMEM((NCH, CH), jnp.int32),
            pltpu.VMEM((NCH, CH), jnp.int32),
            pltpu.VMEM((CH, 2 * IN), jnp.float32),
            pltpu.VMEM((CH, 128), jnp.float32),
            pltpu.VMEM((CH, WID), jnp.float32),
            pltpu.VMEM_SHARED((N, WID), jnp.float32),
            pltpu.SemaphoreType.DMA,
        ],
    )
    def sk(g_hbm, dst2_hbm, src2_hbm, table_hbm, zeros_hbm, out_hbm,
           didx2_v, sidx2_v, xrows_v, g_v, st_v, acc_sh, sem):
        c = lax.axis_index("c")
        s = lax.axis_index("s")
        w = s * NC + c
        iota = lax.iota(jnp.int32, 16)
        ones = jnp.ones((16,), jnp.float32)

        pltpu.sync_copy(dst2_hbm.at[pl.ds(w * NCH, NCH)], didx2_v)
        pltpu.sync_copy(src2_hbm.at[pl.ds(w * NCH, NCH)], sidx2_v)
        pltpu.sync_copy(zeros_hbm.at[pl.ds(s * NPT, NPT)],
                        acc_sh.at[pl.ds(s * NPT, NPT)])

        def pre(r, carry):
            st_v[r, pl.ds(OUT, OUT)] = ones
            return carry

        lax.fori_loop(0, CH, pre, 0)
        plsc.subcore_barrier()

        UNROLL = 4

        def chunk(ci, carry):
            base = w * EPW + ci * CH
            pltpu.async_copy(table_hbm.at[didx2_v.at[ci]],
                             xrows_v, sem).wait()
            pltpu.sync_copy(g_hbm.at[pl.ds(base, CH)], g_v)

            def edge(j, ecarry):
                for u in range(UNROLL):
                    e = j * UNROLL + u
                    xv = xrows_v[e, pl.ds(0, 2 * IN)]
                    a = xv[0] * g_v[e, pl.ds(0, OUT)]
                    for i in range(1, IN):
                        a = a + xv[i] * g_v[e, pl.ds(i * OUT, OUT)]
                    st_v[e, pl.ds(0, OUT)] = a
                return ecarry

            lax.fori_loop(0, CH // UNROLL, edge, 0)
            pltpu.sync_copy(st_v, acc_sh.at[sidx2_v.at[ci]], add=True)
            return carry

        lax.fori_loop(0, NCH, chunk, 0)
        plsc.subcore_barrier()
        pltpu.sync_copy(acc_sh.at[pl.ds(s * NPT, NPT)],
                        out_hbm.at[c, pl.ds(s * NPT, NPT)])

    return sk(g, dst, src2, table, zeros)


def _dott(at, b):
    return lax.dot_general(at, b, dimension_numbers=(((0,), (0,)), ((), ())),
                           preferred_element_type=jnp.float32)


def _tc_main_body(sht_ref, eat_ref, w1_ref, b1_ref, w2_ref, b2_ref,
                  rs_ref, out_ref):
    ea = eat_ref[...].astype(jnp.bfloat16)
    h = jnp.maximum(_dott(ea, w1_ref[...]) + b1_ref[...], 0.0)
    w = jnp.dot(h.astype(jnp.bfloat16), w2_ref[...],
                preferred_element_type=jnp.float32) + b2_ref[...]
    p = _dott(sht_ref[...], rs_ref[...]) * w
    t = p[:, :256] + p[:, 256:]
    out_ref[...] = (t[:, :128] + t[:, 128:]) * SCALE


def _tc_main(sht, eat, w1, b1, w2p, b2p):
    nb = E // BE
    fixed = lambda i: (0, 0)
    return pl.pallas_call(
        _tc_main_body,
        grid=(nb,),
        in_specs=[
            pl.BlockSpec((SH, BE), lambda i: (0, i)),
            pl.BlockSpec((F, BE), lambda i: (0, i)),
            pl.BlockSpec((F, H), fixed),
            pl.BlockSpec((1, H), fixed),
            pl.BlockSpec((H, K * OUT), fixed),
            pl.BlockSpec((1, K * OUT), fixed),
            pl.BlockSpec((SH, K * OUT), fixed),
        ],
        out_specs=pl.BlockSpec((BE, 128), lambda i: (i, 0)),
        out_shape=jax.ShapeDtypeStruct((E, 128), jnp.float32),
    )(sht, eat, w1.astype(jnp.bfloat16), b1, w2p.astype(jnp.bfloat16), b2p,
      jnp.asarray(_RS2))


def _tc_norm_body(a0_ref, a1_ref, na_ref, g_ref, b_ref, out_ref):
    a = a0_ref[...] + a1_ref[...]
    sums = a[:, :OUT]
    cnt = a[:, OUT:OUT + 1]
    o = sums / jnp.maximum(cnt, 1.0) + na_ref[...]
    mean = jnp.mean(o, axis=0, keepdims=True)
    var = jnp.mean((o - mean) ** 2, axis=0, keepdims=True)
    out_ref[...] = (o - mean) * lax.rsqrt(var + 1e-5) * g_ref[...] + b_ref[...]


def _tc_norm(a0, a1, na16, gamma, beta):
    return pl.pallas_call(
        _tc_norm_body,
        out_shape=jax.ShapeDtypeStruct((N, OUT), jnp.float32),
    )(a0, a1, na16, gamma, beta)


def kernel(node_attr, edge_index, edge_attr, edge_sh, W1, b1, W2, b2, gamma, beta):
    na16 = jnp.pad(node_attr, ((0, 0), (0, 2 * IN - IN)))
    src2 = edge_index[0].reshape(E // CH, CH)
    dst2 = edge_index[1].reshape(E // CH, CH)
    W2p = W2.reshape(H, IN, SH, OUT).transpose(0, 2, 1, 3).reshape(H, K * OUT)
    b2p = b2.reshape(IN, SH, OUT).transpose(1, 0, 2).reshape(1, K * OUT)
    g = _tc_main(edge_sh.T, edge_attr.T, W1, b1.reshape(1, H), W2p, b2p)
    acc = _sc_tp_scatter(g, dst2, src2, na16, jnp.zeros((N, WID), jnp.float32))
    return _tc_norm(acc[0], acc[1], na16, gamma.reshape(1, OUT),
                    beta.reshape(1, OUT))


# speedup vs baseline: 1.2247x; 1.2247x over previous
import functools

import numpy as np
import jax
import jax.numpy as jnp
from jax import lax
from jax.experimental import pallas as pl
from jax.experimental.pallas import tpu as pltpu
from jax.experimental.pallas import tpu_sc as plsc

N = 10000
E = 160000
IN = 8
SH = 4
OUT = 16
F = 64
H = 64
K = IN * SH
WID = 2 * OUT
SCALE = 1.0 / float(np.sqrt(IN * SH))

NC, NS = 2, 16
NW = NC * NS
EPW = E // NW
NPT = N // NS
CH = 500
NCH = EPW // CH
NGR = (CH + 15) // 16

BE = 1280

_c = np.arange(K * OUT)
_RS2 = ((_c // 128)[None, :] == np.arange(SH)[:, None]).astype(np.float32)


def _sc_tp_scatter(g, dst, src2, table, zeros):
    mesh = plsc.VectorSubcoreMesh(core_axis_name="c", subcore_axis_name="s")

    @functools.partial(
        pl.kernel, mesh=mesh,
        out_type=jax.ShapeDtypeStruct((NC, N, WID), jnp.float32),
        compiler_params=pltpu.CompilerParams(use_tc_tiling_on_sc=False,
                                             needs_layout_passes=False),
        scratch_types=[
            pltpu.VMEM((NCH, CH), jnp.int32),
            pltpu.VMEM((NCH, CH), jnp.int32),
            pltpu.VMEM((CH, 2 * IN), jnp.float32),
            pltpu.VMEM((CH, 128), jnp.float32),
            pltpu.VMEM((CH, WID), jnp.float32),
            pltpu.VMEM_SHARED((N, WID), jnp.float32),
            pltpu.SemaphoreType.DMA,
        ],
    )
    def sk(g_hbm, dst2_hbm, src2_hbm, table_hbm, zeros_hbm, out_hbm,
           didx2_v, sidx2_v, xrows_v, g_v, st_v, acc_sh, sem):
        c = lax.axis_index("c")
        s = lax.axis_index("s")
        w = s * NC + c
        iota = lax.iota(jnp.int32, 16)
        ones = jnp.ones((16,), jnp.float32)

        pltpu.sync_copy(dst2_hbm.at[pl.ds(w * NCH, NCH)], didx2_v)
        pltpu.sync_copy(src2_hbm.at[pl.ds(w * NCH, NCH)], sidx2_v)
        pltpu.sync_copy(zeros_hbm.at[pl.ds(s * NPT, NPT)],
                        acc_sh.at[pl.ds(s * NPT, NPT)])

        def pre(r, carry):
            st_v[r, pl.ds(OUT, OUT)] = ones
            return carry

        lax.fori_loop(0, CH, pre, 0)
        plsc.subcore_barrier()

        UNROLL = 4

        def chunk(ci, carry):
            base = w * EPW + ci * CH
            pltpu.async_copy(table_hbm.at[didx2_v.at[ci]],
                             xrows_v, sem).wait()
            pltpu.sync_copy(g_hbm.at[pl.ds(base, CH)], g_v)

            def edge(j, ecarry):
                for u in range(UNROLL):
                    e = j * UNROLL + u
                    xv = xrows_v[e, pl.ds(0, 2 * IN)]
                    a = xv[0] * g_v[e, pl.ds(0, OUT)]
                    for i in range(1, IN):
                        a = a + xv[i] * g_v[e, pl.ds(i * OUT, OUT)]
                    st_v[e, pl.ds(0, OUT)] = a
                return ecarry

            lax.fori_loop(0, CH // UNROLL, edge, 0)
            pltpu.sync_copy(st_v, acc_sh.at[sidx2_v.at[ci]], add=True)
            return carry

        lax.fori_loop(0, NCH, chunk, 0)
        plsc.subcore_barrier()
        pltpu.sync_copy(acc_sh.at[pl.ds(s * NPT, NPT)],
                        out_hbm.at[c, pl.ds(s * NPT, NPT)])

    return sk(g, dst, src2, table, zeros)


def _dott(at, b):
    return lax.dot_general(at, b, dimension_numbers=(((0,), (0,)), ((), ())),
                           preferred_element_type=jnp.float32)


def _tc_main_body(sht_ref, eat_ref, w1_ref, b1_ref, w2_ref, b2_ref,
                  rs_ref, out_ref):
    ea = eat_ref[...].astype(jnp.bfloat16)
    h = jnp.maximum(_dott(ea, w1_ref[...]) + b1_ref[...], 0.0)
    w = jnp.dot(h.astype(jnp.bfloat16), w2_ref[...],
                preferred_element_type=jnp.float32) + b2_ref[...]
    p = _dott(sht_ref[...], rs_ref[...]) * w
    t = p[:, :256] + p[:, 256:]
    out_ref[...] = (t[:, :128] + t[:, 128:]) * SCALE


def _tc_main(sht, eat, w1, b1, w2p, b2p):
    nb = E // BE
    fixed = lambda i: (0, 0)
    return pl.pallas_call(
        _tc_main_body,
        grid=(nb,),
        in_specs=[
            pl.BlockSpec((SH, BE), lambda i: (0, i)),
            pl.BlockSpec((F, BE), lambda i: (0, i)),
            pl.BlockSpec((F, H), fixed),
            pl.BlockSpec((1, H), fixed),
            pl.BlockSpec((H, K * OUT), fixed),
            pl.BlockSpec((1, K * OUT), fixed),
            pl.BlockSpec((SH, K * OUT), fixed),
        ],
        out_specs=pl.BlockSpec((BE, 128), lambda i: (i, 0)),
        out_shape=jax.ShapeDtypeStruct((E, 128), jnp.float32),
    )(sht, eat, w1.astype(jnp.bfloat16), b1, w2p.astype(jnp.bfloat16), b2p,
      jnp.asarray(_RS2))


def _tc_norm_body(a0_ref, a1_ref, na_ref, g_ref, b_ref, out_ref):
    a = a0_ref[...] + a1_ref[...]
    sums = a[:, :OUT]
    cnt = a[:, OUT:OUT + 1]
    o = sums / jnp.maximum(cnt, 1.0) + na_ref[...]
    mean = jnp.mean(o, axis=0, keepdims=True)
    var = jnp.mean((o - mean) ** 2, axis=0, keepdims=True)
    out_ref[...] = (o - mean) * lax.rsqrt(var + 1e-5) * g_ref[...] + b_ref[...]


def _tc_norm(a0, a1, na16, gamma, beta):
    return pl.pallas_call(
        _tc_norm_body,
        out_shape=jax.ShapeDtypeStruct((N, OUT), jnp.float32),
    )(a0, a1, na16, gamma, beta)


def kernel(node_attr, edge_index, edge_attr, edge_sh, W1, b1, W2, b2, gamma, beta):
    na16 = jnp.pad(node_attr, ((0, 0), (0, 2 * IN - IN)))
    src2 = edge_index[0].reshape(E // CH, CH)
    dst2 = edge_index[1].reshape(E // CH, CH)
    W2p = W2.reshape(H, IN, SH, OUT).transpose(0, 2, 1, 3).reshape(H, K * OUT)
    b2p = b2.reshape(IN, SH, OUT).transpose(1, 0, 2).reshape(1, K * OUT)
    g = _tc_main(edge_sh.T, edge_attr.T, W1, b1.reshape(1, H), W2p, b2p)
    acc = _sc_tp_scatter(g, dst2, src2, na16, jnp.zeros((N, WID), jnp.float32))
    return _tc_norm(acc[0], acc[1], na16, gamma.reshape(1, OUT),
                    beta.reshape(1, OUT))


# speedup vs baseline: 1.3524x; 1.1043x over previous
import functools

import numpy as np
import jax
import jax.numpy as jnp
from jax import lax
from jax.experimental import pallas as pl
from jax.experimental.pallas import tpu as pltpu
from jax.experimental.pallas import tpu_sc as plsc

N = 10000
E = 160000
IN = 8
SH = 4
OUT = 16
F = 64
H = 64
K = IN * SH
WID = 2 * OUT
SCALE = 1.0 / float(np.sqrt(IN * SH))

NC, NS = 2, 16
NW = NC * NS
EPW = E // NW
NPT = N // NS
CH = 500
NCH = EPW // CH
NGR = (CH + 15) // 16

BE = 3200

_c = np.arange(K * OUT)
_RS2 = ((_c // 128)[None, :] == np.arange(SH)[:, None]).astype(np.float32)


def _sc_tp_scatter(g, dst, src2, table, zeros):
    mesh = plsc.VectorSubcoreMesh(core_axis_name="c", subcore_axis_name="s")

    @functools.partial(
        pl.kernel, mesh=mesh,
        out_type=jax.ShapeDtypeStruct((NC, N, WID), jnp.float32),
        compiler_params=pltpu.CompilerParams(use_tc_tiling_on_sc=False,
                                             needs_layout_passes=False),
        scratch_types=[
            pltpu.VMEM((NCH, CH), jnp.int32),
            pltpu.VMEM((NCH, CH), jnp.int32),
            pltpu.VMEM((CH, 2 * IN), jnp.float32),
            pltpu.VMEM((CH, 128), jnp.float32),
            pltpu.VMEM((CH, WID), jnp.float32),
            pltpu.VMEM_SHARED((N, WID), jnp.float32),
            pltpu.SemaphoreType.DMA,
        ],
    )
    def sk(g_hbm, dst2_hbm, src2_hbm, table_hbm, zeros_hbm, out_hbm,
           didx2_v, sidx2_v, xrows_v, g_v, st_v, acc_sh, sem):
        c = lax.axis_index("c")
        s = lax.axis_index("s")
        w = s * NC + c
        iota = lax.iota(jnp.int32, 16)
        ones = jnp.ones((16,), jnp.float32)

        pltpu.sync_copy(dst2_hbm.at[pl.ds(w * NCH, NCH)], didx2_v)
        pltpu.sync_copy(src2_hbm.at[pl.ds(w * NCH, NCH)], sidx2_v)
        pltpu.sync_copy(zeros_hbm.at[pl.ds(s * NPT, NPT)],
                        acc_sh.at[pl.ds(s * NPT, NPT)])

        def pre(r, carry):
            st_v[r, pl.ds(OUT, OUT)] = ones
            return carry

        lax.fori_loop(0, CH, pre, 0)
        plsc.subcore_barrier()

        UNROLL = 4

        def chunk(ci, carry):
            base = w * EPW + ci * CH
            pltpu.async_copy(table_hbm.at[didx2_v.at[ci]],
                             xrows_v, sem).wait()
            pltpu.sync_copy(g_hbm.at[pl.ds(base, CH)], g_v)

            def edge(j, ecarry):
                for u in range(UNROLL):
                    e = j * UNROLL + u
                    xv = xrows_v[e, pl.ds(0, 2 * IN)]
                    a = xv[0] * g_v[e, pl.ds(0, OUT)]
                    for i in range(1, IN):
                        a = a + xv[i] * g_v[e, pl.ds(i * OUT, OUT)]
                    st_v[e, pl.ds(0, OUT)] = a
                return ecarry

            lax.fori_loop(0, CH // UNROLL, edge, 0)
            pltpu.sync_copy(st_v, acc_sh.at[sidx2_v.at[ci]], add=True)
            return carry

        lax.fori_loop(0, NCH, chunk, 0)
        plsc.subcore_barrier()
        pltpu.sync_copy(acc_sh.at[pl.ds(s * NPT, NPT)],
                        out_hbm.at[c, pl.ds(s * NPT, NPT)])

    return sk(g, dst, src2, table, zeros)


def _dott(at, b):
    return lax.dot_general(at, b, dimension_numbers=(((0,), (0,)), ((), ())),
                           preferred_element_type=jnp.float32)


def _tc_main_body(sht_ref, eat_ref, w1_ref, b1_ref, w2_ref, b2_ref,
                  rs_ref, out_ref):
    ea = eat_ref[...].astype(jnp.bfloat16)
    h = jnp.maximum(_dott(ea, w1_ref[...]) + b1_ref[...], 0.0)
    w = jnp.dot(h.astype(jnp.bfloat16), w2_ref[...],
                preferred_element_type=jnp.float32) + b2_ref[...]
    p = _dott(sht_ref[...], rs_ref[...]) * w
    t = p[:, :256] + p[:, 256:]
    out_ref[...] = (t[:, :128] + t[:, 128:]) * SCALE


def _tc_main(sht, eat, w1, b1, w2p, b2p):
    nb = E // BE
    fixed = lambda i: (0, 0)
    return pl.pallas_call(
        _tc_main_body,
        grid=(nb,),
        in_specs=[
            pl.BlockSpec((SH, BE), lambda i: (0, i)),
            pl.BlockSpec((F, BE), lambda i: (0, i)),
            pl.BlockSpec((F, H), fixed),
            pl.BlockSpec((1, H), fixed),
            pl.BlockSpec((H, K * OUT), fixed),
            pl.BlockSpec((1, K * OUT), fixed),
            pl.BlockSpec((SH, K * OUT), fixed),
        ],
        out_specs=pl.BlockSpec((BE, 128), lambda i: (i, 0)),
        out_shape=jax.ShapeDtypeStruct((E, 128), jnp.float32),
    )(sht, eat, w1.astype(jnp.bfloat16), b1, w2p.astype(jnp.bfloat16), b2p,
      jnp.asarray(_RS2))


def _tc_norm_body(a0_ref, a1_ref, na_ref, g_ref, b_ref, out_ref):
    a = a0_ref[...] + a1_ref[...]
    sums = a[:, :OUT]
    cnt = a[:, OUT:OUT + 1]
    o = sums / jnp.maximum(cnt, 1.0) + na_ref[...]
    mean = jnp.mean(o, axis=0, keepdims=True)
    var = jnp.mean((o - mean) ** 2, axis=0, keepdims=True)
    out_ref[...] = (o - mean) * lax.rsqrt(var + 1e-5) * g_ref[...] + b_ref[...]


def _tc_norm(a0, a1, na16, gamma, beta):
    return pl.pallas_call(
        _tc_norm_body,
        out_shape=jax.ShapeDtypeStruct((N, OUT), jnp.float32),
    )(a0, a1, na16, gamma, beta)


def kernel(node_attr, edge_index, edge_attr, edge_sh, W1, b1, W2, b2, gamma, beta):
    na16 = jnp.pad(node_attr, ((0, 0), (0, 2 * IN - IN)))
    src2 = edge_index[0].reshape(E // CH, CH)
    dst2 = edge_index[1].reshape(E // CH, CH)
    W2p = W2.reshape(H, IN, SH, OUT).transpose(0, 2, 1, 3).reshape(H, K * OUT)
    b2p = b2.reshape(IN, SH, OUT).transpose(1, 0, 2).reshape(1, K * OUT)
    g = _tc_main(edge_sh.T, edge_attr.T, W1, b1.reshape(1, H), W2p, b2p)
    acc = _sc_tp_scatter(g, dst2, src2, na16, jnp.zeros((N, WID), jnp.float32))
    return _tc_norm(acc[0], acc[1], na16, gamma.reshape(1, OUT),
                    beta.reshape(1, OUT))


# speedup vs baseline: 1.3836x; 1.0230x over previous
import functools

import numpy as np
import jax
import jax.numpy as jnp
from jax import lax
from jax.experimental import pallas as pl
from jax.experimental.pallas import tpu as pltpu
from jax.experimental.pallas import tpu_sc as plsc

N = 10000
E = 160000
IN = 8
SH = 4
OUT = 16
F = 64
H = 64
K = IN * SH
WID = 2 * OUT
SCALE = 1.0 / float(np.sqrt(IN * SH))

NC, NS = 2, 16
NW = NC * NS
EPW = E // NW
NPT = N // NS
CH = 500
NCH = EPW // CH
NGR = (CH + 15) // 16

BE = 6400

_c = np.arange(K * OUT)
_RS2 = ((_c // 128)[None, :] == np.arange(SH)[:, None]).astype(np.float32)


def _sc_tp_scatter(g, dst, src2, table, zeros):
    mesh = plsc.VectorSubcoreMesh(core_axis_name="c", subcore_axis_name="s")

    @functools.partial(
        pl.kernel, mesh=mesh,
        out_type=jax.ShapeDtypeStruct((NC, N, WID), jnp.float32),
        compiler_params=pltpu.CompilerParams(use_tc_tiling_on_sc=False,
                                             needs_layout_passes=False),
        scratch_types=[
            pltpu.VMEM((NCH, CH), jnp.int32),
            pltpu.VMEM((NCH, CH), jnp.int32),
            pltpu.VMEM((CH, 2 * IN), jnp.float32),
            pltpu.VMEM((CH, 128), jnp.float32),
            pltpu.VMEM((CH, WID), jnp.float32),
            pltpu.VMEM_SHARED((N, WID), jnp.float32),
            pltpu.SemaphoreType.DMA,
        ],
    )
    def sk(g_hbm, dst2_hbm, src2_hbm, table_hbm, zeros_hbm, out_hbm,
           didx2_v, sidx2_v, xrows_v, g_v, st_v, acc_sh, sem):
        c = lax.axis_index("c")
        s = lax.axis_index("s")
        w = s * NC + c
        iota = lax.iota(jnp.int32, 16)
        ones = jnp.ones((16,), jnp.float32)

        pltpu.sync_copy(dst2_hbm.at[pl.ds(w * NCH, NCH)], didx2_v)
        pltpu.sync_copy(src2_hbm.at[pl.ds(w * NCH, NCH)], sidx2_v)
        pltpu.sync_copy(zeros_hbm.at[pl.ds(s * NPT, NPT)],
                        acc_sh.at[pl.ds(s * NPT, NPT)])

        def pre(r, carry):
            st_v[r, pl.ds(OUT, OUT)] = ones
            return carry

        lax.fori_loop(0, CH, pre, 0)
        plsc.subcore_barrier()

        UNROLL = 4

        def chunk(ci, carry):
            base = w * EPW + ci * CH
            pltpu.async_copy(table_hbm.at[didx2_v.at[ci]],
                             xrows_v, sem).wait()
            pltpu.sync_copy(g_hbm.at[pl.ds(base, CH)], g_v)

            def edge(j, ecarry):
                for u in range(UNROLL):
                    e = j * UNROLL + u
                    xv = xrows_v[e, pl.ds(0, 2 * IN)]
                    a = xv[0] * g_v[e, pl.ds(0, OUT)]
                    for i in range(1, IN):
                        a = a + xv[i] * g_v[e, pl.ds(i * OUT, OUT)]
                    st_v[e, pl.ds(0, OUT)] = a
                return ecarry

            lax.fori_loop(0, CH // UNROLL, edge, 0)
            pltpu.sync_copy(st_v, acc_sh.at[sidx2_v.at[ci]], add=True)
            return carry

        lax.fori_loop(0, NCH, chunk, 0)
        plsc.subcore_barrier()
        pltpu.sync_copy(acc_sh.at[pl.ds(s * NPT, NPT)],
                        out_hbm.at[c, pl.ds(s * NPT, NPT)])

    return sk(g, dst, src2, table, zeros)


def _dott(at, b):
    return lax.dot_general(at, b, dimension_numbers=(((0,), (0,)), ((), ())),
                           preferred_element_type=jnp.float32)


def _tc_main_body(sht_ref, eat_ref, w1_ref, b1_ref, w2_ref, b2_ref,
                  rs_ref, out_ref):
    ea = eat_ref[...].astype(jnp.bfloat16)
    h = jnp.maximum(_dott(ea, w1_ref[...]) + b1_ref[...], 0.0)
    w = jnp.dot(h.astype(jnp.bfloat16), w2_ref[...],
                preferred_element_type=jnp.float32) + b2_ref[...]
    p = _dott(sht_ref[...], rs_ref[...]) * w
    t = p[:, :256] + p[:, 256:]
    out_ref[...] = (t[:, :128] + t[:, 128:]) * SCALE


def _tc_main(sht, eat, w1, b1, w2p, b2p):
    nb = E // BE
    fixed = lambda i: (0, 0)
    return pl.pallas_call(
        _tc_main_body,
        grid=(nb,),
        in_specs=[
            pl.BlockSpec((SH, BE), lambda i: (0, i)),
            pl.BlockSpec((F, BE), lambda i: (0, i)),
            pl.BlockSpec((F, H), fixed),
            pl.BlockSpec((1, H), fixed),
            pl.BlockSpec((H, K * OUT), fixed),
            pl.BlockSpec((1, K * OUT), fixed),
            pl.BlockSpec((SH, K * OUT), fixed),
        ],
        out_specs=pl.BlockSpec((BE, 128), lambda i: (i, 0)),
        out_shape=jax.ShapeDtypeStruct((E, 128), jnp.float32),
    )(sht, eat, w1.astype(jnp.bfloat16), b1, w2p.astype(jnp.bfloat16), b2p,
      jnp.asarray(_RS2))


def _tc_norm_body(a0_ref, a1_ref, na_ref, g_ref, b_ref, out_ref):
    a = a0_ref[...] + a1_ref[...]
    sums = a[:, :OUT]
    cnt = a[:, OUT:OUT + 1]
    o = sums / jnp.maximum(cnt, 1.0) + na_ref[...]
    mean = jnp.mean(o, axis=0, keepdims=True)
    var = jnp.mean((o - mean) ** 2, axis=0, keepdims=True)
    out_ref[...] = (o - mean) * lax.rsqrt(var + 1e-5) * g_ref[...] + b_ref[...]


def _tc_norm(a0, a1, na16, gamma, beta):
    return pl.pallas_call(
        _tc_norm_body,
        out_shape=jax.ShapeDtypeStruct((N, OUT), jnp.float32),
    )(a0, a1, na16, gamma, beta)


def kernel(node_attr, edge_index, edge_attr, edge_sh, W1, b1, W2, b2, gamma, beta):
    na16 = jnp.pad(node_attr, ((0, 0), (0, 2 * IN - IN)))
    src2 = edge_index[0].reshape(E // CH, CH)
    dst2 = edge_index[1].reshape(E // CH, CH)
    W2p = W2.reshape(H, IN, SH, OUT).transpose(0, 2, 1, 3).reshape(H, K * OUT)
    b2p = b2.reshape(IN, SH, OUT).transpose(1, 0, 2).reshape(1, K * OUT)
    g = _tc_main(edge_sh.T, edge_attr.T, W1, b1.reshape(1, H), W2p, b2p)
    acc = _sc_tp_scatter(g, dst2, src2, na16, jnp.zeros((N, WID), jnp.float32))
    return _tc_norm(acc[0], acc[1], na16, gamma.reshape(1, OUT),
                    beta.reshape(1, OUT))


# speedup vs baseline: 1.5544x; 1.1235x over previous
import functools

import numpy as np
import jax
import jax.numpy as jnp
from jax import lax
from jax.experimental import pallas as pl
from jax.experimental.pallas import tpu as pltpu
from jax.experimental.pallas import tpu_sc as plsc

N = 10000
E = 160000
IN = 8
SH = 4
OUT = 16
F = 64
H = 64
K = IN * SH
WID = 2 * OUT
SCALE = 1.0 / float(np.sqrt(IN * SH))

NC, NS = 2, 16
NW = NC * NS
SEG = 2
ES = E // SEG
EPW = ES // NW
NPT = N // NS
CH = 500
NCH = EPW // CH
NCR = E // CH

BE = 3200

_c = np.arange(K * OUT)
_RS2 = ((_c // 128)[None, :] == np.arange(SH)[:, None]).astype(np.float32)


def _sc_tp_scatter(g, dst2, src2, table, zeros, seg):
    mesh = plsc.VectorSubcoreMesh(core_axis_name="c", subcore_axis_name="s")

    @functools.partial(
        pl.kernel, mesh=mesh,
        out_type=jax.ShapeDtypeStruct((NC, N, WID), jnp.float32),
        compiler_params=pltpu.CompilerParams(use_tc_tiling_on_sc=False,
                                             needs_layout_passes=False),
        scratch_types=[
            pltpu.VMEM((NCH, CH), jnp.int32),
            pltpu.VMEM((NCH, CH), jnp.int32),
            pltpu.VMEM((CH, 2 * IN), jnp.float32),
            pltpu.VMEM((CH, 128), jnp.float32),
            pltpu.VMEM((CH, WID), jnp.float32),
            pltpu.VMEM_SHARED((N, WID), jnp.float32),
            pltpu.SemaphoreType.DMA,
        ],
    )
    def sk(g_hbm, dst2_hbm, src2_hbm, table_hbm, zeros_hbm, out_hbm,
           didx2_v, sidx2_v, xrows_v, g_v, st_v, acc_sh, sem):
        c = lax.axis_index("c")
        s = lax.axis_index("s")
        w = s * NC + c
        iota = lax.iota(jnp.int32, 16)
        ones = jnp.ones((16,), jnp.float32)

        row0 = seg * (ES // CH) + w * NCH
        pltpu.sync_copy(dst2_hbm.at[pl.ds(row0, NCH)], didx2_v)
        pltpu.sync_copy(src2_hbm.at[pl.ds(row0, NCH)], sidx2_v)
        pltpu.sync_copy(zeros_hbm.at[pl.ds(s * NPT, NPT)],
                        acc_sh.at[pl.ds(s * NPT, NPT)])

        def pre(r, carry):
            st_v[r, pl.ds(OUT, OUT)] = ones
            return carry

        lax.fori_loop(0, CH, pre, 0)
        plsc.subcore_barrier()

        UNROLL = 4

        def chunk(ci, carry):
            base = w * EPW + ci * CH
            pltpu.async_copy(table_hbm.at[didx2_v.at[ci]],
                             xrows_v, sem).wait()
            pltpu.sync_copy(g_hbm.at[pl.ds(base, CH)], g_v)

            def edge(j, ecarry):
                for u in range(UNROLL):
                    e = j * UNROLL + u
                    xv = xrows_v[e, pl.ds(0, 2 * IN)]
                    a = xv[0] * g_v[e, pl.ds(0, OUT)]
                    for i in range(1, IN):
                        a = a + xv[i] * g_v[e, pl.ds(i * OUT, OUT)]
                    st_v[e, pl.ds(0, OUT)] = a
                return ecarry

            lax.fori_loop(0, CH // UNROLL, edge, 0)
            pltpu.sync_copy(st_v, acc_sh.at[sidx2_v.at[ci]], add=True)
            return carry

        lax.fori_loop(0, NCH, chunk, 0)
        plsc.subcore_barrier()
        pltpu.sync_copy(acc_sh.at[pl.ds(s * NPT, NPT)],
                        out_hbm.at[c, pl.ds(s * NPT, NPT)])

    return sk(g, dst2, src2, table, zeros)


def _dott(at, b):
    return lax.dot_general(at, b, dimension_numbers=(((0,), (0,)), ((), ())),
                           preferred_element_type=jnp.float32)


def _tc_main_body(sht_ref, eat_ref, w1_ref, b1_ref, w2_ref, b2_ref,
                  rs_ref, out_ref):
    ea = eat_ref[...].astype(jnp.bfloat16)
    h = jnp.maximum(_dott(ea, w1_ref[...]) + b1_ref[...], 0.0)
    w = jnp.dot(h.astype(jnp.bfloat16), w2_ref[...],
                preferred_element_type=jnp.float32) + b2_ref[...]
    p = _dott(sht_ref[...], rs_ref[...]) * w
    t = p[:, :256] + p[:, 256:]
    out_ref[...] = (t[:, :128] + t[:, 128:]) * SCALE


def _tc_main(sht, eat, w1, b1, w2p, b2p, seg):
    nb = ES // BE
    off = seg * nb
    fixed = lambda i: (0, 0)
    return pl.pallas_call(
        _tc_main_body,
        grid=(nb,),
        in_specs=[
            pl.BlockSpec((SH, BE), lambda i: (0, i + off)),
            pl.BlockSpec((F, BE), lambda i: (0, i + off)),
            pl.BlockSpec((F, H), fixed),
            pl.BlockSpec((1, H), fixed),
            pl.BlockSpec((H, K * OUT), fixed),
            pl.BlockSpec((1, K * OUT), fixed),
            pl.BlockSpec((SH, K * OUT), fixed),
        ],
        out_specs=pl.BlockSpec((BE, 128), lambda i: (i, 0)),
        out_shape=jax.ShapeDtypeStruct((ES, 128), jnp.float32),
    )(sht, eat, w1.astype(jnp.bfloat16), b1, w2p.astype(jnp.bfloat16), b2p,
      jnp.asarray(_RS2))


def _tc_norm_body(a0_ref, a1_ref, na_ref, g_ref, b_ref, out_ref):
    a = (a0_ref[0] + a0_ref[1]) + (a1_ref[0] + a1_ref[1])
    sums = a[:, :OUT]
    cnt = a[:, OUT:OUT + 1]
    o = sums / jnp.maximum(cnt, 1.0) + na_ref[...]
    mean = jnp.mean(o, axis=0, keepdims=True)
    var = jnp.mean((o - mean) ** 2, axis=0, keepdims=True)
    out_ref[...] = (o - mean) * lax.rsqrt(var + 1e-5) * g_ref[...] + b_ref[...]


def _tc_norm(a0, a1, na16, gamma, beta):
    return pl.pallas_call(
        _tc_norm_body,
        out_shape=jax.ShapeDtypeStruct((N, OUT), jnp.float32),
    )(a0, a1, na16, gamma, beta)


def kernel(node_attr, edge_index, edge_attr, edge_sh, W1, b1, W2, b2, gamma, beta):
    na16 = jnp.pad(node_attr, ((0, 0), (0, 2 * IN - IN)))
    src2 = edge_index[0].reshape(E // CH, CH)
    dst2 = edge_index[1].reshape(E // CH, CH)
    W2p = W2.reshape(H, IN, SH, OUT).transpose(0, 2, 1, 3).reshape(H, K * OUT)
    b2p = b2.reshape(IN, SH, OUT).transpose(1, 0, 2).reshape(1, K * OUT)
    zeros = jnp.zeros((N, WID), jnp.float32)
    accs = []
    for seg in range(SEG):
        g = _tc_main(edge_sh.T, edge_attr.T, W1, b1.reshape(1, H), W2p, b2p,
                     seg)
        accs.append(_sc_tp_scatter(g, dst2, src2, na16, zeros, seg))
    return _tc_norm(accs[0], accs[1], na16, gamma.reshape(1, OUT),
                    beta.reshape(1, OUT))
